# baseline (speedup 1.0000x reference)
import jax
import jax.numpy as jnp
from jax import lax
from jax.experimental import pallas as pl
from jax.experimental.pallas import tpu as pltpu
from jax.experimental.pallas import tpu_sc as plsc

B = 4096
L = 200
EMBED = 64
VOCAB = 1000000
_NSTEPS = VOCAB // 256
_VPART = _NSTEPS * 256
_L0 = 112
_L1 = 96

_info = plsc.get_sparse_core_info()
_NC, _NS = _info.num_cores, _info.num_subcores
_NW = _NC * _NS
_SEQ_PER_W = B // _NW

_mesh = plsc.VectorSubcoreMesh(core_axis_name="c", subcore_axis_name="s")
_params = pltpu.CompilerParams(use_tc_tiling_on_sc=True,
                               needs_layout_passes=False)


def _worker_id():
    return lax.axis_index("s") * _NC + lax.axis_index("c")


def _transpose_body(tok_hbm, tail_hbm, stg_hbm, in_vm, tout_vm, isem, osem):
    wid = _worker_id()
    iota = lax.iota(jnp.int32, 16)
    nb = (_NSTEPS - wid + _NW - 1) // _NW

    def step_id(k):
        return wid + k * _NW

    def issue_in(k, h):
        for jb in range(8):
            pltpu.async_copy(
                tok_hbm.at[pl.ds(jb * 8, 8), pl.ds(step_id(k) * 256, 256)],
                in_vm.at[h, pl.ds(jb * 8, 8)], isem.at[h])

    def wait_in(k, h):
        for jb in range(8):
            pltpu.make_async_copy(
                tok_hbm.at[pl.ds(jb * 8, 8), pl.ds(step_id(k) * 256, 256)],
                in_vm.at[h, pl.ds(jb * 8, 8)], isem.at[h]).wait()

    def issue_out(k, h):
        pltpu.async_copy(tout_vm.at[h],
                         stg_hbm.at[pl.ds(step_id(k) * 128, 128)], osem.at[h])

    def wait_out(h):
        pltpu.make_async_copy(tout_vm.at[h], stg_hbm.at[pl.ds(0, 128)],
                              osem.at[h]).wait()

    rows = [tb * 8 + (iota >> 1) for tb in range(16)]
    col0 = (iota & 1) * EMBED

    def transpose(h):
        @plsc.parallel_loop(0, EMBED, unroll=4)
        def jbody(j):
            cols = col0 + j
            for tb in range(16):
                v = in_vm[h, j, pl.ds(tb * 16, 16)]
                plsc.store_scatter(tout_vm.at[h], [rows[tb], cols], v)

    def step(k, h):
        @pl.when(k + 1 < nb)
        def _():
            issue_in(k + 1, h ^ 1)
        wait_in(k, h)

        @pl.when(k >= 2)
        def _():
            wait_out(h)
        issue_out(k, h)

    issue_in(0, 0)

    def pair(g, c):
        for h in range(2):
            k = g * 2 + h

            @pl.when(k < nb)
            def _(k=k, h=h):
                step(k, h)
        return c

    lax.fori_loop(0, (_NSTEPS + 2 * _NW - 1) // (2 * _NW), pair, 0)
    wait_out(0)
    wait_out(1)

    @pl.when(wid == 0)
    def _():
        pltpu.sync_copy(tail_hbm, tout_vm.at[0, pl.ds(0, 32)])
        pltpu.sync_copy(tout_vm.at[0, pl.ds(0, 32)],
                        stg_hbm.at[pl.ds(_VPART // 2, 32)])


def _gather_body(x_hbm, possp_hbm, stg_hbm, out_hbm,
                 xv, fidx, pbufv, oidx, pos_pg, rows_v, out_vm,
                 gsem, psem, osem):
    wid = _worker_id()
    base = wid * _SEQ_PER_W
    iota = lax.iota(jnp.int32, 16)

    pltpu.sync_copy(x_hbm.at[:, pl.ds(base, _SEQ_PER_W)], xv)

    def prep(l, b):
        for bb in range(8):
            v = xv[l, pl.ds(bb * 16, 16)]
            fidx[b, pl.ds(bb * 16, 16)] = (v >> 1) * 0
            pbufv[b, pl.ds(bb * 16, 16)] = (v & 1) << 6
        ebase = l * (EMBED * _NW) + wid
        for eb in range(4):
            oidx[b, pl.ds(eb * 16, 16)] = \
                ebase + (eb * 16 + iota) * _NW
        pltpu.async_copy(possp_hbm.at[l], pos_pg.at[b], psem.at[b])

    def issue_gather(b):
        pltpu.async_copy(stg_hbm.at[fidx.at[b]], rows_v.at[b], gsem.at[b])

    def wait_gather(b):
        pltpu.make_async_copy(stg_hbm.at[fidx.at[b]], rows_v.at[b],
                              gsem.at[b]).wait()
        pltpu.make_async_copy(possp_hbm.at[0], pos_pg.at[b],
                              psem.at[b]).wait()

    def issue_out(l, b):
        pltpu.async_copy(out_vm.at[b], out_hbm.at[oidx.at[b]], osem.at[b])

    def wait_out(b):
        pltpu.make_async_copy(out_vm.at[b], out_hbm.at[oidx.at[b]],
                              osem.at[b]).wait()

    def compute(l, b):
        pvs = [pbufv[b, pl.ds(bb * 16, 16)] for bb in range(8)]
        bvs = [bb * 16 + iota for bb in range(8)]

        @plsc.parallel_loop(0, EMBED, unroll=4)
        def ebody(e):
            pbc = pos_pg[b, pl.ds(e * 16, 16)]
            for bb in range(8):
                col = pvs[bb] + e
                val = plsc.load_gather(rows_v.at[b], [bvs[bb], col])
                out_vm[b, e, pl.ds(bb * 16, 16)] = val + pbc

    def halfstep(l, b):
        @pl.when(l + 2 < L)
        def _():
            prep(l + 2, (b + 2) % 4)
            issue_gather((b + 2) % 4)
        wait_gather(b)

        @pl.when(l >= 4)
        def _():
            wait_out(b)
        compute(l, b)
        issue_out(l, b)

    prep(0, 0)
    issue_gather(0)
    prep(1, 1)
    issue_gather(1)

    def grp(g, c):
        for h in range(4):
            halfstep(4 * g + h, h)
        return c

    lax.fori_loop(0, L // 4, grp, 0)
    for h in range(4):
        wait_out(h)


def kernel(x, tok_table, pos_table):
    xT = x.astype(jnp.int32).T
    tokT = tok_table.T
    pos_splat = jnp.broadcast_to(
        pos_table[:L, :, None], (L, EMBED, 16)).reshape(L, EMBED * 16)
    tail = tok_table[_VPART:].reshape(VOCAB // 2 - _VPART // 2, 128)

    stage1 = pl.kernel(
        _transpose_body,
        mesh=_mesh,
        out_type=jax.ShapeDtypeStruct((VOCAB // 2, 128), jnp.float32),
        scratch_types=[
            pltpu.VMEM((2, EMBED, 256), jnp.float32),
            pltpu.VMEM((2, 128, 128), jnp.float32),
            pltpu.SemaphoreType.DMA((2,)),
            pltpu.SemaphoreType.DMA((2,)),
        ],
        compiler_params=_params,
    )
    staging = stage1(tokT, tail)

    stage2 = pl.kernel(
        _gather_body,
        mesh=_mesh,
        out_type=jax.ShapeDtypeStruct((L * EMBED * _NW, 128), jnp.float32),
        scratch_types=[
            pltpu.VMEM((L, _SEQ_PER_W), jnp.int32),
            pltpu.VMEM((4, _SEQ_PER_W), jnp.int32),
            pltpu.VMEM((4, _SEQ_PER_W), jnp.int32),
            pltpu.VMEM((4, EMBED), jnp.int32),
            pltpu.VMEM((4, EMBED * 16), jnp.float32),
            pltpu.VMEM((4, _SEQ_PER_W, 128), jnp.float32),
            pltpu.VMEM((4, EMBED, _SEQ_PER_W), jnp.float32),
            pltpu.SemaphoreType.DMA((4,)),
            pltpu.SemaphoreType.DMA((4,)),
            pltpu.SemaphoreType.DMA((4,)),
        ],
        compiler_params=_params,
    )
    outP = stage2(xT, pos_splat, staging)
    outP = outP.reshape(L, EMBED, B)
    return jnp.transpose(outP, (2, 0, 1))


# speedup vs baseline: 18.7351x; 18.7351x over previous
import jax
import jax.numpy as jnp
from jax import lax
from jax.experimental import pallas as pl
from jax.experimental.pallas import tpu as pltpu
from jax.experimental.pallas import tpu_sc as plsc

B = 4096
L = 200
EMBED = 64
VOCAB = 1000000
_NSTEPS = VOCAB // 128
_VPART = _NSTEPS * 128
_L0 = 112
_L1 = 96

_info = plsc.get_sparse_core_info()
_NC, _NS = _info.num_cores, _info.num_subcores
_NW = _NC * _NS
_SEQ_PER_W = B // _NW

_mesh = plsc.VectorSubcoreMesh(core_axis_name="c", subcore_axis_name="s")
_params = pltpu.CompilerParams(use_tc_tiling_on_sc=True,
                               needs_layout_passes=False)


def _worker_id():
    return lax.axis_index("s") * _NC + lax.axis_index("c")


def _transpose_body(tok_hbm, tail_hbm, stg_hbm, in_vm, tout_vm, isem, osem):
    wid = _worker_id()
    iota = lax.iota(jnp.int32, 16)
    nb = (_NSTEPS - wid + _NW - 1) // _NW

    def step_id(k):
        return wid + k * _NW

    def issue_in(k, h):
        for jb in range(8):
            pltpu.async_copy(
                tok_hbm.at[pl.ds(jb * 8, 8), pl.ds(step_id(k) * 128, 128)],
                in_vm.at[h, pl.ds(jb * 8, 8)], isem.at[h])

    def wait_in(k, h):
        for jb in range(8):
            pltpu.make_async_copy(
                tok_hbm.at[pl.ds(jb * 8, 8), pl.ds(step_id(k) * 128, 128)],
                in_vm.at[h, pl.ds(jb * 8, 8)], isem.at[h]).wait()

    def issue_out(k, h):
        pltpu.async_copy(tout_vm.at[h],
                         stg_hbm.at[pl.ds(step_id(k) * 64, 64)], osem.at[h])

    def wait_out(h):
        pltpu.make_async_copy(tout_vm.at[h], stg_hbm.at[pl.ds(0, 64)],
                              osem.at[h]).wait()

    rowv = [(cb * 16 + iota) & 63 for cb in range(8)]
    colv = [(cb * 16 + iota) >> 6 for cb in range(8)]

    def transpose(h):
        @plsc.parallel_loop(0, EMBED, unroll=4)
        def fbody(f):
            for cb in range(8):
                v = plsc.load_gather(in_vm.at[h], [rowv[cb], colv[cb] + 2 * f])
                tout_vm[h, f, pl.ds(cb * 16, 16)] = v

    def step(k, h):
        @pl.when(k + 1 < nb)
        def _():
            issue_in(k + 1, h ^ 1)
        wait_in(k, h)

        @pl.when(k >= 2)
        def _():
            wait_out(h)
        transpose(h)
        issue_out(k, h)

    issue_in(0, 0)

    def pair(g, c):
        for h in range(2):
            k = g * 2 + h

            @pl.when(k < nb)
            def _(k=k, h=h):
                step(k, h)
        return c

    lax.fori_loop(0, (_NSTEPS + 2 * _NW - 1) // (2 * _NW), pair, 0)
    wait_out(0)
    wait_out(1)

    @pl.when(wid == 0)
    def _():
        pltpu.sync_copy(tail_hbm, tout_vm.at[0, pl.ds(0, 32)])
        pltpu.sync_copy(tout_vm.at[0, pl.ds(0, 32)],
                        stg_hbm.at[pl.ds(_VPART // 2, 32)])


def _gather_body(x_hbm, possp_hbm, stg_hbm, out_hbm,
                 xv, fidx, pbufv, oidx, pos_pg, rows_v, out_vm,
                 gsem, psem, osem):
    wid = _worker_id()
    base = wid * _SEQ_PER_W
    iota = lax.iota(jnp.int32, 16)

    pltpu.sync_copy(x_hbm.at[:, pl.ds(base, _SEQ_PER_W)], xv)

    def prep(l, b):
        for bb in range(8):
            v = xv[l, pl.ds(bb * 16, 16)]
            fidx[b, bb // 4, pl.ds((bb % 4) * 16, 16)] = v >> 1
            pbufv[b, pl.ds(bb * 16, 16)] = (v & 1) << 6
        ebase = l * (EMBED * _NW) + wid
        for eb in range(4):
            oidx[b, pl.ds(eb * 16, 16)] = \
                ebase + (eb * 16 + iota) * _NW
        pltpu.async_copy(possp_hbm.at[l], pos_pg.at[b], psem.at[b])

    def issue_gather(b):
        pltpu.async_copy(stg_hbm.at[fidx.at[b, 0]],
                         rows_v.at[b, pl.ds(0, 64)], gsem.at[b])
        pltpu.async_copy(stg_hbm.at[fidx.at[b, 1]],
                         rows_v.at[b, pl.ds(64, 64)], gsem.at[b])

    def wait_gather(b):
        pltpu.make_async_copy(stg_hbm.at[fidx.at[b, 0]],
                              rows_v.at[b, pl.ds(0, 64)], gsem.at[b]).wait()
        pltpu.make_async_copy(stg_hbm.at[fidx.at[b, 1]],
                              rows_v.at[b, pl.ds(64, 64)], gsem.at[b]).wait()
        pltpu.make_async_copy(possp_hbm.at[0], pos_pg.at[b],
                              psem.at[b]).wait()

    def issue_out(l, b):
        pltpu.async_copy(out_vm.at[b], out_hbm.at[oidx.at[b]], osem.at[b])

    def wait_out(b):
        pltpu.make_async_copy(out_vm.at[b], out_hbm.at[oidx.at[b]],
                              osem.at[b]).wait()

    def compute(l, b):
        pvs = [pbufv[b, pl.ds(bb * 16, 16)] for bb in range(8)]
        bvs = [bb * 16 + iota for bb in range(8)]

        @plsc.parallel_loop(0, EMBED, unroll=4)
        def ebody(e):
            pbc = pos_pg[b, pl.ds(e * 16, 16)]
            for bb in range(8):
                col = pvs[bb] + e
                val = plsc.load_gather(rows_v.at[b], [bvs[bb], col])
                out_vm[b, e, pl.ds(bb * 16, 16)] = val + pbc

    def halfstep(l, b):
        @pl.when(l + 3 < L)
        def _():
            prep(l + 3, (b + 3) % 4)
            issue_gather((b + 3) % 4)
        wait_gather(b)

        @pl.when(l >= 4)
        def _():
            wait_out(b)
        compute(l, b)
        issue_out(l, b)

    for l0 in range(3):
        prep(l0, l0)
        issue_gather(l0)

    def grp(g, c):
        for h in range(4):
            halfstep(4 * g + h, h)
        return c

    lax.fori_loop(0, L // 4, grp, 0)
    for h in range(4):
        wait_out(h)


def kernel(x, tok_table, pos_table):
    xT = x.astype(jnp.int32).T
    tokT = tok_table.T
    pos_splat = jnp.broadcast_to(
        pos_table[:L, :, None], (L, EMBED, 16)).reshape(L, EMBED * 16)
    tail = tok_table[_VPART:].reshape(VOCAB // 2 - _VPART // 2, 128)

    stage1 = pl.kernel(
        _transpose_body,
        mesh=_mesh,
        out_type=jax.ShapeDtypeStruct((VOCAB // 2, 128), jnp.float32),
        scratch_types=[
            pltpu.VMEM((2, EMBED, 128), jnp.float32),
            pltpu.VMEM((2, EMBED, 128), jnp.float32),
            pltpu.SemaphoreType.DMA((2,)),
            pltpu.SemaphoreType.DMA((2,)),
        ],
        compiler_params=_params,
    )
    staging = stage1(tokT, tail)

    stage2 = pl.kernel(
        _gather_body,
        mesh=_mesh,
        out_type=jax.ShapeDtypeStruct((L * EMBED * _NW, 128), jnp.float32),
        scratch_types=[
            pltpu.VMEM((L, _SEQ_PER_W), jnp.int32),
            pltpu.VMEM((4, 2, 64), jnp.int32),
            pltpu.VMEM((4, _SEQ_PER_W), jnp.int32),
            pltpu.VMEM((4, EMBED), jnp.int32),
            pltpu.VMEM((4, EMBED * 16), jnp.float32),
            pltpu.VMEM((4, _SEQ_PER_W, 128), jnp.float32),
            pltpu.VMEM((4, EMBED, _SEQ_PER_W), jnp.float32),
            pltpu.SemaphoreType.DMA((4,)),
            pltpu.SemaphoreType.DMA((4,)),
            pltpu.SemaphoreType.DMA((4,)),
        ],
        compiler_params=_params,
    )
    outP = stage2(xT, pos_splat, staging)
    outP = outP.reshape(L, EMBED, B)
    return jnp.transpose(outP, (2, 0, 1))


# speedup vs baseline: 18.7681x; 1.0018x over previous
import jax
import jax.numpy as jnp
from jax import lax
from jax.experimental import pallas as pl
from jax.experimental.pallas import tpu as pltpu
from jax.experimental.pallas import tpu_sc as plsc

B = 4096
L = 200
EMBED = 64
VOCAB = 1000000
_NSTEPS = VOCAB // 128
_VPART = _NSTEPS * 128
_L0 = 112
_L1 = 96

_info = plsc.get_sparse_core_info()
_NC, _NS = _info.num_cores, _info.num_subcores
_NW = _NC * _NS
_SEQ_PER_W = B // _NW

_mesh = plsc.VectorSubcoreMesh(core_axis_name="c", subcore_axis_name="s")
_params = pltpu.CompilerParams(use_tc_tiling_on_sc=True,
                               needs_layout_passes=False)


def _worker_id():
    return lax.axis_index("s") * _NC + lax.axis_index("c")


def _transpose_body(tok_hbm, tail_hbm, stg_hbm, in_vm, tout_vm, isem, osem):
    wid = _worker_id()
    iota = lax.iota(jnp.int32, 16)
    nb = (_NSTEPS - wid + _NW - 1) // _NW

    def step_id(k):
        return wid + k * _NW

    def issue_in(k, h):
        for jb in range(8):
            pltpu.async_copy(
                tok_hbm.at[pl.ds(jb * 8, 8), pl.ds(step_id(k) * 128, 128)],
                in_vm.at[h, pl.ds(jb * 8, 8)], isem.at[h])

    def wait_in(k, h):
        for jb in range(8):
            pltpu.make_async_copy(
                tok_hbm.at[pl.ds(jb * 8, 8), pl.ds(step_id(k) * 128, 128)],
                in_vm.at[h, pl.ds(jb * 8, 8)], isem.at[h]).wait()

    def issue_out(k, h):
        pltpu.async_copy(tout_vm.at[h],
                         stg_hbm.at[pl.ds(step_id(k) * 64, 64)], osem.at[h])

    def wait_out(h):
        pltpu.make_async_copy(tout_vm.at[h], stg_hbm.at[pl.ds(0, 64)],
                              osem.at[h]).wait()

    rowv = [(cb * 16 + iota) & 63 for cb in range(8)]
    colv = [(cb * 16 + iota) >> 6 for cb in range(8)]

    def transpose(h):
        @plsc.parallel_loop(0, EMBED, unroll=4)
        def fbody(f):
            for cb in range(8):
                v = plsc.load_gather(in_vm.at[h], [rowv[cb], colv[cb] + 2 * f])
                tout_vm[h, f, pl.ds(cb * 16, 16)] = v

    def step(k, h):
        @pl.when(k + 1 < nb)
        def _():
            issue_in(k + 1, h ^ 1)
        wait_in(k, h)

        @pl.when(k >= 2)
        def _():
            wait_out(h)
        transpose(h)
        issue_out(k, h)

    issue_in(0, 0)

    def pair(g, c):
        for h in range(2):
            k = g * 2 + h

            @pl.when(k < nb)
            def _(k=k, h=h):
                step(k, h)
        return c

    lax.fori_loop(0, (_NSTEPS + 2 * _NW - 1) // (2 * _NW), pair, 0)
    wait_out(0)
    wait_out(1)

    @pl.when(wid == 0)
    def _():
        pltpu.sync_copy(tail_hbm, tout_vm.at[0, pl.ds(0, 32)])
        pltpu.sync_copy(tout_vm.at[0, pl.ds(0, 32)],
                        stg_hbm.at[pl.ds(_VPART // 2, 32)])


def _gather_body(x_hbm, possp_hbm, stg_hbm, out_hbm,
                 xv, fidx, pbufv, oidx, pos_pg, rows_v, out_vm,
                 gsem, psem, osem):
    wid = _worker_id()
    base = wid * _SEQ_PER_W
    iota = lax.iota(jnp.int32, 16)

    pltpu.sync_copy(x_hbm.at[:, pl.ds(base, _SEQ_PER_W)], xv)

    def prep(l, b):
        for bb in range(8):
            v = xv[l, pl.ds(bb * 16, 16)]
            fidx[b, pl.ds(bb * 16, 16)] = v >> 1
            pbufv[b, pl.ds(bb * 16, 16)] = (v & 1) << 6
        ebase = l * (EMBED * _NW) + wid
        for eb in range(4):
            oidx[b, pl.ds(eb * 16, 16)] = \
                ebase + (eb * 16 + iota) * _NW
        pltpu.async_copy(possp_hbm.at[l], pos_pg.at[b], psem.at[b])

    def issue_gather(b):
        pltpu.async_copy(stg_hbm.at[fidx.at[b]], rows_v.at[b], gsem.at[b])

    def wait_gather(b):
        pltpu.make_async_copy(stg_hbm.at[fidx.at[b]], rows_v.at[b],
                              gsem.at[b]).wait()
        pltpu.make_async_copy(possp_hbm.at[0], pos_pg.at[b],
                              psem.at[b]).wait()

    def issue_out(l, b):
        pltpu.async_copy(out_vm.at[b], out_hbm.at[oidx.at[b]], osem.at[b])

    def wait_out(b):
        pltpu.make_async_copy(out_vm.at[b], out_hbm.at[oidx.at[b]],
                              osem.at[b]).wait()

    def compute(l, b):
        pvs = [pbufv[b, pl.ds(bb * 16, 16)] for bb in range(8)]
        bvs = [bb * 16 + iota for bb in range(8)]

        @plsc.parallel_loop(0, EMBED, unroll=4)
        def ebody(e):
            pbc = pos_pg[b, pl.ds(e * 16, 16)]
            for bb in range(8):
                col = pvs[bb] + e
                val = plsc.load_gather(rows_v.at[b], [bvs[bb], col])
                out_vm[b, e, pl.ds(bb * 16, 16)] = val + pbc

    def halfstep(l, b):
        @pl.when(l + 3 < L)
        def _():
            prep(l + 3, (b + 3) % 4)
            issue_gather((b + 3) % 4)
        wait_gather(b)

        @pl.when(l >= 4)
        def _():
            wait_out(b)
        compute(l, b)
        issue_out(l, b)

    for l0 in range(3):
        prep(l0, l0)
        issue_gather(l0)

    def grp(g, c):
        for h in range(4):
            halfstep(4 * g + h, h)
        return c

    lax.fori_loop(0, L // 4, grp, 0)
    for h in range(4):
        wait_out(h)


def kernel(x, tok_table, pos_table):
    xT = x.astype(jnp.int32).T
    tokT = tok_table.T
    pos_splat = jnp.broadcast_to(
        pos_table[:L, :, None], (L, EMBED, 16)).reshape(L, EMBED * 16)
    tail = tok_table[_VPART:].reshape(VOCAB // 2 - _VPART // 2, 128)

    stage1 = pl.kernel(
        _transpose_body,
        mesh=_mesh,
        out_type=jax.ShapeDtypeStruct((VOCAB // 2, 128), jnp.float32),
        scratch_types=[
            pltpu.VMEM((2, EMBED, 128), jnp.float32),
            pltpu.VMEM((2, EMBED, 128), jnp.float32),
            pltpu.SemaphoreType.DMA((2,)),
            pltpu.SemaphoreType.DMA((2,)),
        ],
        compiler_params=_params,
    )
    staging = stage1(tokT, tail)

    stage2 = pl.kernel(
        _gather_body,
        mesh=_mesh,
        out_type=jax.ShapeDtypeStruct((L * EMBED * _NW, 128), jnp.float32),
        scratch_types=[
            pltpu.VMEM((L, _SEQ_PER_W), jnp.int32),
            pltpu.VMEM((4, _SEQ_PER_W), jnp.int32),
            pltpu.VMEM((4, _SEQ_PER_W), jnp.int32),
            pltpu.VMEM((4, EMBED), jnp.int32),
            pltpu.VMEM((4, EMBED * 16), jnp.float32),
            pltpu.VMEM((4, _SEQ_PER_W, 128), jnp.float32),
            pltpu.VMEM((4, EMBED, _SEQ_PER_W), jnp.float32),
            pltpu.SemaphoreType.DMA((4,)),
            pltpu.SemaphoreType.DMA((4,)),
            pltpu.SemaphoreType.DMA((4,)),
        ],
        compiler_params=_params,
    )
    outP = stage2(xT, pos_splat, staging)
    outP = outP.reshape(L, EMBED, B)
    return jnp.transpose(outP, (2, 0, 1))


# speedup vs baseline: 18.7982x; 1.0016x over previous
import jax
import jax.numpy as jnp
from jax import lax
from jax.experimental import pallas as pl
from jax.experimental.pallas import tpu as pltpu
from jax.experimental.pallas import tpu_sc as plsc

B = 4096
L = 200
EMBED = 64
VOCAB = 1000000
_NSTEPS = VOCAB // 128
_VPART = _NSTEPS * 128
_L0 = 112
_L1 = 96

_info = plsc.get_sparse_core_info()
_NC, _NS = _info.num_cores, _info.num_subcores
_NW = _NC * _NS
_SEQ_PER_W = B // _NW

_mesh = plsc.VectorSubcoreMesh(core_axis_name="c", subcore_axis_name="s")
_params = pltpu.CompilerParams(use_tc_tiling_on_sc=True,
                               needs_layout_passes=False)


def _worker_id():
    return lax.axis_index("s") * _NC + lax.axis_index("c")


def _transpose_body(tok_hbm, tail_hbm, stg_hbm, in_vm, tout_vm, isem, osem):
    wid = _worker_id()
    iota = lax.iota(jnp.int32, 16)
    nb = (_NSTEPS - wid + _NW - 1) // _NW

    def step_id(k):
        return wid + k * _NW

    def issue_in(k, h):
        for jb in range(8):
            pltpu.async_copy(
                tok_hbm.at[pl.ds(jb * 8, 8), pl.ds(step_id(k) * 128, 128)],
                in_vm.at[h, pl.ds(jb * 8, 8)], isem.at[h])

    def wait_in(k, h):
        for jb in range(8):
            pltpu.make_async_copy(
                tok_hbm.at[pl.ds(jb * 8, 8), pl.ds(step_id(k) * 128, 128)],
                in_vm.at[h, pl.ds(jb * 8, 8)], isem.at[h]).wait()

    def issue_out(k, h):
        pltpu.async_copy(tout_vm.at[h],
                         stg_hbm.at[pl.ds(step_id(k) * 64, 64)], osem.at[h])

    def wait_out(h):
        pltpu.make_async_copy(tout_vm.at[h], stg_hbm.at[pl.ds(0, 64)],
                              osem.at[h]).wait()

    rowv = [(cb * 16 + iota) & 63 for cb in range(8)]
    colv = [(cb * 16 + iota) >> 6 for cb in range(8)]

    def transpose(h):
        @plsc.parallel_loop(0, EMBED, unroll=4)
        def fbody(f):
            for cb in range(8):
                v = plsc.load_gather(in_vm.at[h], [rowv[cb], colv[cb] + 2 * f])
                tout_vm[h, f, pl.ds(cb * 16, 16)] = v

    def step(k, h):
        @pl.when(k + 1 < nb)
        def _():
            issue_in(k + 1, h ^ 1)
        wait_in(k, h)

        @pl.when(k >= 2)
        def _():
            wait_out(h)
        transpose(h)
        issue_out(k, h)

    issue_in(0, 0)

    def pair(g, c):
        for h in range(2):
            k = g * 2 + h

            @pl.when(k < nb)
            def _(k=k, h=h):
                step(k, h)
        return c

    lax.fori_loop(0, (_NSTEPS + 2 * _NW - 1) // (2 * _NW), pair, 0)
    wait_out(0)
    wait_out(1)

    @pl.when(wid == 0)
    def _():
        pltpu.sync_copy(tail_hbm, tout_vm.at[0, pl.ds(0, 32)])
        pltpu.sync_copy(tout_vm.at[0, pl.ds(0, 32)],
                        stg_hbm.at[pl.ds(_VPART // 2, 32)])


def _gather_body(x_hbm, possp_hbm, stg_hbm, out_hbm,
                 xv, fidx, pbufv, oidx, pos_pg, rows_v, out_vm,
                 gsem, psem, osem):
    wid = _worker_id()
    base = wid * _SEQ_PER_W
    iota = lax.iota(jnp.int32, 16)

    pltpu.sync_copy(x_hbm.at[:, pl.ds(base, _SEQ_PER_W)], xv)

    def prep(l, b):
        for bb in range(8):
            v = xv[l, pl.ds(bb * 16, 16)]
            fidx[b, pl.ds(bb * 16, 16)] = v >> 1
            pbufv[b, pl.ds(bb * 16, 16)] = (v & 1) << 6
        ebase = l * (EMBED * _NW) + wid
        for eb in range(4):
            oidx[b, pl.ds(eb * 16, 16)] = \
                ebase + (eb * 16 + iota) * _NW
        pltpu.async_copy(possp_hbm.at[l], pos_pg.at[b], psem.at[b])

    def issue_gather(b):
        pltpu.async_copy(stg_hbm.at[fidx.at[b]], rows_v.at[b], gsem.at[b])

    def wait_gather(b):
        pltpu.make_async_copy(stg_hbm.at[fidx.at[b]], rows_v.at[b],
                              gsem.at[b]).wait()
        pltpu.make_async_copy(possp_hbm.at[0], pos_pg.at[b],
                              psem.at[b]).wait()

    def issue_out(l, b):
        pltpu.async_copy(out_vm.at[b], out_hbm.at[oidx.at[b]], osem.at[b])

    def wait_out(b):
        pltpu.make_async_copy(out_vm.at[b], out_hbm.at[oidx.at[b]],
                              osem.at[b]).wait()

    def compute(l, b):
        pvs = [pbufv[b, pl.ds(bb * 16, 16)] for bb in range(8)]
        bvs = [bb * 16 + iota for bb in range(8)]

        @plsc.parallel_loop(0, EMBED, unroll=4)
        def ebody(e):
            pbc = pos_pg[b, pl.ds(e * 16, 16)]
            for bb in range(8):
                col = pvs[bb] + e
                val = plsc.load_gather(rows_v.at[b], [bvs[bb], col])
                out_vm[b, e, pl.ds(bb * 16, 16)] = val + pbc

    def halfstep(l, b):
        @pl.when(l + 2 < L)
        def _():
            @pl.when(l >= 2)
            def _():
                wait_out((b + 2) % 4)
            prep(l + 2, (b + 2) % 4)
            issue_gather((b + 2) % 4)
        wait_gather(b)
        compute(l, b)
        issue_out(l, b)

    for l0 in range(2):
        prep(l0, l0)
        issue_gather(l0)

    def grp(g, c):
        for h in range(4):
            halfstep(4 * g + h, h)
        return c

    lax.fori_loop(0, L // 4, grp, 0)
    for h in range(4):
        wait_out(h)


def kernel(x, tok_table, pos_table):
    xT = x.astype(jnp.int32).T
    tokT = tok_table.T
    pos_splat = jnp.broadcast_to(
        pos_table[:L, :, None], (L, EMBED, 16)).reshape(L, EMBED * 16)
    tail = tok_table[_VPART:].reshape(VOCAB // 2 - _VPART // 2, 128)

    stage1 = pl.kernel(
        _transpose_body,
        mesh=_mesh,
        out_type=jax.ShapeDtypeStruct((VOCAB // 2, 128), jnp.float32),
        scratch_types=[
            pltpu.VMEM((2, EMBED, 128), jnp.float32),
            pltpu.VMEM((2, EMBED, 128), jnp.float32),
            pltpu.SemaphoreType.DMA((2,)),
            pltpu.SemaphoreType.DMA((2,)),
        ],
        compiler_params=_params,
    )
    staging = stage1(tokT, tail)

    stage2 = pl.kernel(
        _gather_body,
        mesh=_mesh,
        out_type=jax.ShapeDtypeStruct((L * EMBED * _NW, 128), jnp.float32),
        scratch_types=[
            pltpu.VMEM((L, _SEQ_PER_W), jnp.int32),
            pltpu.VMEM((4, _SEQ_PER_W), jnp.int32),
            pltpu.VMEM((4, _SEQ_PER_W), jnp.int32),
            pltpu.VMEM((4, EMBED), jnp.int32),
            pltpu.VMEM((4, EMBED * 16), jnp.float32),
            pltpu.VMEM((4, _SEQ_PER_W, 128), jnp.float32),
            pltpu.VMEM((4, EMBED, _SEQ_PER_W), jnp.float32),
            pltpu.SemaphoreType.DMA((4,)),
            pltpu.SemaphoreType.DMA((4,)),
            pltpu.SemaphoreType.DMA((4,)),
        ],
        compiler_params=_params,
    )
    outP = stage2(xT, pos_splat, staging)
    outP = outP.reshape(L, EMBED, B)
    return jnp.transpose(outP, (2, 0, 1))


# speedup vs baseline: 25.4965x; 1.3563x over previous
import jax
import jax.numpy as jnp
from jax import lax
from jax.experimental import pallas as pl
from jax.experimental.pallas import tpu as pltpu
from jax.experimental.pallas import tpu_sc as plsc

B = 4096
L = 200
EMBED = 64
_HALF = L // 2
_NBUF = 4
_LOOK = 2

_info = plsc.get_sparse_core_info()
_NC, _NS = _info.num_cores, _info.num_subcores
_NW = _NC * _NS
_SEQ_PER_W = B // _NW


def _embed_body(x_hbm, tok_hbm, pos_hbm, out_hbm, idx_v, pos_v, rows_v,
                gsem, osem):
    wid = lax.axis_index("s") * _NC + lax.axis_index("c")
    base = wid * _SEQ_PER_W
    pltpu.sync_copy(x_hbm.at[pl.ds(base, _SEQ_PER_W)], idx_v)
    pltpu.sync_copy(pos_hbm.at[pl.ds(0, L)], pos_v)

    def issue_gather(s, b):
        pltpu.async_copy(tok_hbm.at[idx_v.at[s, 0]],
                         rows_v.at[b, pl.ds(0, _HALF)], gsem.at[b])
        pltpu.async_copy(tok_hbm.at[idx_v.at[s, 1]],
                         rows_v.at[b, pl.ds(_HALF, _HALF)], gsem.at[b])

    def wait_gather(s, b):
        pltpu.make_async_copy(tok_hbm.at[idx_v.at[s, 0]],
                              rows_v.at[b, pl.ds(0, _HALF)], gsem.at[b]).wait()
        pltpu.make_async_copy(tok_hbm.at[idx_v.at[s, 1]],
                              rows_v.at[b, pl.ds(_HALF, _HALF)],
                              gsem.at[b]).wait()

    def issue_out(s, b):
        pltpu.async_copy(rows_v.at[b], out_hbm.at[base + s], osem.at[b])

    def wait_out(b):
        pltpu.make_async_copy(rows_v.at[b], out_hbm.at[base], osem.at[b]).wait()

    def compute(b):
        def add_rows(r4, c):
            for u in range(4):
                r = r4 * 4 + u
                for d in range(EMBED // 16):
                    sl = pl.ds(d * 16, 16)
                    rows_v[b, r, sl] = rows_v[b, r, sl] + pos_v[r, sl]
            return c
        lax.fori_loop(0, L // 4, add_rows, 0)

    def step(s, b, skip_out_wait):
        nb = (b + _LOOK) % _NBUF
        if skip_out_wait:
            issue_gather(s + _LOOK, nb)
        else:
            @pl.when(s + _LOOK < _SEQ_PER_W)
            def _():
                wait_out(nb)
                issue_gather(s + _LOOK, nb)
        wait_gather(s, b)
        compute(b)
        issue_out(s, b)

    issue_gather(0, 0)
    issue_gather(1, 1)
    for b in range(_NBUF):
        step(b, b, skip_out_wait=(b < _LOOK))

    def group(g, c):
        for b in range(_NBUF):
            step(g * _NBUF + b, b, skip_out_wait=False)
        return c

    lax.fori_loop(1, _SEQ_PER_W // _NBUF, group, 0)

    for b in range(_NBUF):
        wait_out(b)


def kernel(x, tok_table, pos_table):
    x3 = x.astype(jnp.int32).reshape(B, 2, _HALF)
    mesh = plsc.VectorSubcoreMesh(core_axis_name="c", subcore_axis_name="s")
    f = pl.kernel(
        _embed_body,
        mesh=mesh,
        out_type=jax.ShapeDtypeStruct((B, L, EMBED), jnp.float32),
        scratch_types=[
            pltpu.VMEM((_SEQ_PER_W, 2, _HALF), jnp.int32),
            pltpu.VMEM((L, EMBED), jnp.float32),
            pltpu.VMEM((_NBUF, L, EMBED), jnp.float32),
            pltpu.SemaphoreType.DMA((_NBUF,)),
            pltpu.SemaphoreType.DMA((_NBUF,)),
        ],
        compiler_params=pltpu.CompilerParams(use_tc_tiling_on_sc=False),
    )
    return f(x3, tok_table, pos_table)
